# Initial kernel scaffold; baseline (speedup 1.0000x reference)
import jax
import jax.numpy as jnp
from jax.experimental import pallas as pl


def kernel(x, tables):
    raise NotImplementedError("write your pallas kernel here")


# speedup vs baseline: 2.1653x; 2.1653x over previous
import functools

import numpy as np
import jax
import jax.numpy as jnp
from jax import lax
from jax.experimental import pallas as pl
from jax.experimental.pallas import tpu as pltpu
from jax.experimental.pallas import tpu_sc as plsc

_B, _F, _S, _D, _V, _C = 1024, 26, 50, 32, 100000, 32
_NC, _NS = 2, 16
_NW = _NC * _NS
_BPW = _B // _NW
_XW = _C * _S
_NCAT = _F * _S
_ROWS = 1312
_FT = _D * _S
_CAT = _F * _FT
_OUTW = _CAT + (_C - _F) * _S
_CONT0 = _F * _S


def _consts():
    off = np.zeros((_ROWS,), np.int32)
    off[:_NCAT] = (np.arange(_NCAT) // _S).astype(np.int32) * _V
    return off


_OFF = _consts()


def _emb_body(x_hbm, tab_hbm, off_hbm, out_hbm,
              x_v, off_v, idx_v, rows_v, out_v, sem):
    cid = lax.axis_index("c")
    sid = lax.axis_index("s")
    wid = sid * _NC + cid

    pltpu.sync_copy(off_hbm, off_v)
    dvec = lax.broadcasted_iota(jnp.int32, (16,), 0) * _S

    def per_b(i, carry):
        b = wid * _BPW + i
        pltpu.sync_copy(x_hbm.at[b], x_v.at[pl.ds(0, _XW)])

        def idx_body(j, c2):
            v = x_v[pl.ds(j * 16, 16)]
            iv = (v + 0.5).astype(jnp.int32)
            iv = jnp.minimum(jnp.maximum(iv, 0), _V - 1)
            idx_v[pl.ds(j * 16, 16)] = iv + off_v[pl.ds(j * 16, 16)]
            return c2
        lax.fori_loop(0, _ROWS // 16, idx_body, 0, unroll=4)

        pltpu.async_copy(tab_hbm.at[idx_v], rows_v, sem).wait()

        def f_body(f, c2):
            def s_body(s, c3):
                r = f * _S + s
                v0 = rows_v[r, pl.ds(0, 16)]
                v1 = rows_v[r, pl.ds(16, 16)]
                d0 = dvec + (f * _FT + s)
                plsc.store_scatter(out_v, [d0], v0)
                plsc.store_scatter(out_v, [d0 + 16 * _S], v1)
                return c3
            return lax.fori_loop(0, _S, s_body, c2, unroll=5)
        lax.fori_loop(0, _F, f_body, 0)

        def c_body(j, c2):
            src = lax.broadcasted_iota(jnp.int32, (16,), 0) + (_CONT0 + j * 16)
            vals = plsc.load_gather(x_v, [src])
            dst = lax.broadcasted_iota(jnp.int32, (16,), 0) + (_CAT + j * 16)
            plsc.store_scatter(out_v, [dst], vals, mask=dst < _OUTW)
            return c2
        lax.fori_loop(0, 19, c_body, 0)

        pltpu.sync_copy(out_v, out_hbm.at[b])
        return carry

    lax.fori_loop(0, _BPW, per_b, 0)


@functools.partial(jax.jit, static_argnums=())
def _run(x2, tab2, off):
    mesh = plsc.VectorSubcoreMesh(core_axis_name="c", subcore_axis_name="s",
                                  num_cores=_NC, num_subcores=_NS)
    f = pl.kernel(
        _emb_body,
        out_type=jax.ShapeDtypeStruct((_B, _OUTW), jnp.float32),
        mesh=mesh,
        compiler_params=pltpu.CompilerParams(needs_layout_passes=False,
                                             use_tc_tiling_on_sc=False),
        scratch_types=[
            pltpu.VMEM((_XW + 16,), jnp.float32),
            pltpu.VMEM((_ROWS,), jnp.int32),
            pltpu.VMEM((_ROWS,), jnp.int32),
            pltpu.VMEM((_ROWS, _D), jnp.float32),
            pltpu.VMEM((_OUTW,), jnp.float32),
            pltpu.SemaphoreType.DMA,
        ],
    )
    return f(x2, tab2, off)


def kernel(x, tables):
    x2 = x.reshape(_B, _XW)
    tab2 = tables.reshape(_F * _V, _D)
    out = _run(x2, tab2, jnp.asarray(_OFF))
    return out.reshape(_B, _F * _D + (_C - _F), _S)


# speedup vs baseline: 2.2648x; 1.0459x over previous
import functools

import numpy as np
import jax
import jax.numpy as jnp
from jax import lax
from jax.experimental import pallas as pl
from jax.experimental.pallas import tpu as pltpu
from jax.experimental.pallas import tpu_sc as plsc

_B, _F, _S, _D, _V, _C = 1024, 26, 50, 32, 100000, 32
_NC, _NS = 2, 16
_NW = _NC * _NS
_BPW = _B // _NW
_XW = _C * _S
_NCAT = _F * _S
_ROWS = 1312
_FT = _D * _S
_CAT = _F * _FT
_OUTW = _CAT + (_C - _F) * _S
_CONT0 = _F * _S
_CONTW = (_C - _F) * _S
_HF = 13
_HROWS = 656
_H1OFF = 648
_H1LO = _HF * _S - _H1OFF
_HW0 = _HF * _FT
_HW1 = _HF * _FT + _CONTW
_OBW = 21104


def _off_const():
    off = np.zeros((_ROWS,), np.int32)
    off[:_NCAT] = (np.arange(_NCAT) // _S).astype(np.int32) * _V
    return off


_OFF = _off_const()


def _compute_idx(x_ref, off_ref, idx_ref):
    def body(j, c):
        v = x_ref[pl.ds(j * 16, 16)]
        iv = (v + 0.5).astype(jnp.int32)
        iv = jnp.minimum(jnp.maximum(iv, 0), _V - 1)
        idx_ref[pl.ds(j * 16, 16)] = iv + off_ref[pl.ds(j * 16, 16)]
        return c
    lax.fori_loop(0, _ROWS // 16, body, 0, unroll=4)


def _transpose_half(rows_ref, ob_ref, lo, dvec):
    def f_body(g, c):
        def s_body(s, c2):
            r = g * _S + s + lo
            v0 = rows_ref[r, pl.ds(0, 16)]
            v1 = rows_ref[r, pl.ds(16, 16)]
            d0 = dvec + (g * _FT + s)
            plsc.store_scatter(ob_ref, [d0], v0)
            plsc.store_scatter(ob_ref, [d0 + 16 * _S], v1)
            return c2
        return lax.fori_loop(0, _S, s_body, c, unroll=5)
    lax.fori_loop(0, _HF, f_body, 0)


def _emb_body(x_hbm, tab_hbm, off_hbm, out_hbm,
              x0, x1, off_v, i0, i1, r0, r1, o0, o1,
              semx, semg0, semg1, semo0, semo1):
    cid = lax.axis_index("c")
    sid = lax.axis_index("s")
    wid = sid * _NC + cid
    b0 = wid * _BPW

    pltpu.sync_copy(off_hbm, off_v)
    dvec = lax.broadcasted_iota(jnp.int32, (16,), 0) * _S
    xs = (x0, x1)
    idxs = (i0, i1)

    def fire_g0(idx_ref):
        return pltpu.async_copy(
            tab_hbm.at[idx_ref.at[pl.ds(0, _HROWS)]], r0, semg0)

    def fire_g1(idx_ref):
        return pltpu.async_copy(
            tab_hbm.at[idx_ref.at[pl.ds(_H1OFF, _HROWS)]], r1, semg1)

    def fire_x(b, x_ref):
        return pltpu.async_copy(x_hbm.at[b], x_ref.at[pl.ds(0, _XW)], semx)

    def wait_x(b, x_ref):
        pltpu.make_async_copy(x_hbm.at[b], x_ref.at[pl.ds(0, _XW)],
                              semx).wait()

    def wait_g0(idx_ref):
        pltpu.make_async_copy(
            tab_hbm.at[idx_ref.at[pl.ds(0, _HROWS)]], r0, semg0).wait()

    def wait_g1(idx_ref):
        pltpu.make_async_copy(
            tab_hbm.at[idx_ref.at[pl.ds(_H1OFF, _HROWS)]], r1, semg1).wait()

    def fire_o0(b):
        return pltpu.async_copy(o0.at[pl.ds(0, _HW0)],
                                out_hbm.at[b, pl.ds(0, _HW0)], semo0)

    def fire_o1(b):
        return pltpu.async_copy(o1.at[pl.ds(0, _HW1)],
                                out_hbm.at[b, pl.ds(_HW0, _HW1)], semo1)

    def drain_o0(b):
        pltpu.make_async_copy(o0.at[pl.ds(0, _HW0)],
                              out_hbm.at[b, pl.ds(0, _HW0)], semo0).wait()

    def drain_o1(b):
        pltpu.make_async_copy(o1.at[pl.ds(0, _HW1)],
                              out_hbm.at[b, pl.ds(_HW0, _HW1)], semo1).wait()

    def cont_extract(x_ref):
        def body(j, c):
            src = lax.broadcasted_iota(jnp.int32, (16,), 0) + (
                _CONT0 + j * 16)
            vals = plsc.load_gather(x_ref, [src])
            o1[pl.ds(_HW0 + j * 16, 16)] = vals
            return c
        lax.fori_loop(0, (_OBW - _HW0) // 16, body, 0)

    pltpu.sync_copy(x_hbm.at[b0], x0.at[pl.ds(0, _XW)])
    _compute_idx(x0, off_v, i0)
    fire_g0(i0)
    fire_x(b0 + 1, x1)

    def per_b(u, t, q):
        b = b0 + t
        xq, xn = xs[q], xs[1 - q]
        iq, inx = idxs[q], idxs[1 - q]

        fire_g1(iq)

        if q == 0:
            wait_x(b + 1, xn)
            _compute_idx(xn, off_v, inx)
        else:
            @pl.when(u < _BPW // 2 - 1)
            def _():
                wait_x(b + 1, xn)
                _compute_idx(xn, off_v, inx)

        wait_g0(iq)
        if q == 0:
            @pl.when(u > 0)
            def _():
                drain_o0(b - 1)
        else:
            drain_o0(b - 1)
        _transpose_half(r0, o0, 0, dvec)
        fire_o0(b)

        if q == 0:
            fire_g0(inx)
        else:
            @pl.when(u < _BPW // 2 - 1)
            def _():
                fire_g0(inx)

        if q == 0:
            @pl.when(u > 0)
            def _():
                drain_o1(b - 1)
        else:
            drain_o1(b - 1)
        cont_extract(xq)
        wait_g1(iq)
        _transpose_half(r1, o1, _H1LO, dvec)
        fire_o1(b)

        @pl.when(u < _BPW // 2 - 1)
        def _():
            fire_x(b + 2, xq)

    def pair_body(u, c):
        per_b(u, 2 * u, 0)
        per_b(u, 2 * u + 1, 1)
        return c
    lax.fori_loop(0, _BPW // 2, pair_body, 0)

    drain_o0(b0 + _BPW - 1)
    drain_o1(b0 + _BPW - 1)


@functools.partial(jax.jit, static_argnums=())
def _run(x2, tab2, off):
    mesh = plsc.VectorSubcoreMesh(core_axis_name="c", subcore_axis_name="s",
                                  num_cores=_NC, num_subcores=_NS)
    f = pl.kernel(
        _emb_body,
        out_type=jax.ShapeDtypeStruct((_B, _OUTW), jnp.float32),
        mesh=mesh,
        compiler_params=pltpu.CompilerParams(needs_layout_passes=False,
                                             use_tc_tiling_on_sc=False),
        scratch_types=[
            pltpu.VMEM((_XW + 16,), jnp.float32),
            pltpu.VMEM((_XW + 16,), jnp.float32),
            pltpu.VMEM((_ROWS,), jnp.int32),
            pltpu.VMEM((_ROWS,), jnp.int32),
            pltpu.VMEM((_ROWS,), jnp.int32),
            pltpu.VMEM((_HROWS, _D), jnp.float32),
            pltpu.VMEM((_HROWS, _D), jnp.float32),
            pltpu.VMEM((_OBW,), jnp.float32),
            pltpu.VMEM((_OBW,), jnp.float32),
            pltpu.SemaphoreType.DMA,
            pltpu.SemaphoreType.DMA,
            pltpu.SemaphoreType.DMA,
            pltpu.SemaphoreType.DMA,
            pltpu.SemaphoreType.DMA,
        ],
    )
    return f(x2, tab2, off)


def kernel(x, tables):
    x2 = x.reshape(_B, _XW)
    tab2 = tables.reshape(_F * _V, _D)
    out = _run(x2, tab2, jnp.asarray(_OFF))
    return out.reshape(_B, _F * _D + (_C - _F), _S)
